# baseline (speedup 1.0000x reference)
import jax
import jax.numpy as jnp
from jax import lax
from jax.experimental import pallas as pl
from jax.experimental.pallas import tpu as pltpu

_N = 32768
_C = 256
_G = 32
_CPG = _C // _G
_NI = 64
_EPS = 1e-5
_BLK = 8192
_NBLK = _N // _BLK
_HI = lax.Precision.HIGHEST


def _hilo(v):
    hi = v.astype(jnp.bfloat16)
    lo = (v - hi.astype(jnp.float32)).astype(jnp.bfloat16)
    return hi, lo


def _body(seg_row_ref, seg_col_ref, x_ref, w_ref, b_ref, o_ref,
          s1_acc, s2_acc, cnt_acc, tab, xsave):
    p = pl.program_id(0)
    i = pl.program_id(1)

    @pl.when(p == 0)
    def _stats():
        @pl.when(i == 0)
        def _init():
            s1_acc[...] = jnp.zeros_like(s1_acc)
            s2_acc[...] = jnp.zeros_like(s2_acc)
            cnt_acc[...] = jnp.zeros_like(cnt_acc)

        x = x_ref[...]
        xb = x.astype(jnp.bfloat16)
        xsave[pl.ds(i * _BLK, _BLK), :] = xb
        seg = seg_row_ref[0]
        onehot = (
            lax.broadcasted_iota(jnp.int32, (_NI, _BLK), 0) == seg
        ).astype(jnp.bfloat16)
        s1_acc[...] += jnp.dot(onehot, xb, preferred_element_type=jnp.float32)
        s2_acc[...] += jnp.dot(onehot, xb * xb, preferred_element_type=jnp.float32)
        cnt_acc[...] += jnp.sum(
            onehot.astype(jnp.float32), axis=1, keepdims=True
        )

        @pl.when(i == _NBLK - 1)
        def _epilogue():
            s12 = jnp.concatenate([s1_acc[...], s2_acc[...]], axis=1)
            eg = (
                lax.broadcasted_iota(jnp.int32, (2 * _C, 2 * _G), 0) // _CPG
                == lax.broadcasted_iota(jnp.int32, (2 * _C, 2 * _G), 1)
            ).astype(jnp.float32)
            sg = jnp.dot(s12, eg, precision=_HI)
            cnt = jnp.maximum(cnt_acc[...] * float(_CPG), 1.0)
            mean = sg[:, :_G] / cnt
            var = sg[:, _G:] / cnt - mean * mean
            inv = lax.rsqrt(var + _EPS)
            rg = (
                lax.broadcasted_iota(jnp.int32, (_G, _C), 0)
                == lax.broadcasted_iota(jnp.int32, (_G, _C), 1) // _CPG
            ).astype(jnp.float32)
            inv_c = jnp.dot(inv, rg, precision=_HI)
            mean_c = jnp.dot(mean, rg, precision=_HI)
            scale = inv_c * w_ref[...]
            shift = b_ref[...] - mean_c * scale
            tab[...] = jnp.concatenate(
                [scale.astype(jnp.bfloat16), shift.astype(jnp.bfloat16)], axis=1
            )

    @pl.when(p == 1)
    def _norm():
        segc = seg_col_ref[0]
        onehot = (
            segc == lax.broadcasted_iota(jnp.int32, (_BLK, _NI), 1)
        ).astype(jnp.bfloat16)
        t = tab[...]
        sc_tok = jnp.dot(onehot, t[:, :_C], preferred_element_type=jnp.float32)
        sh_tok = jnp.dot(onehot, t[:, _C:], preferred_element_type=jnp.float32)
        xs = xsave[pl.ds(i * _BLK, _BLK), :].astype(jnp.float32)
        o_ref[...] = xs * sc_tok + sh_tok


def kernel(features, ins_indices_batch, ins_ids, weight, bias):
    del ins_ids
    seg = ins_indices_batch.astype(jnp.int32)
    seg_row = seg.reshape(_NBLK, 1, _BLK)
    seg_col = seg.reshape(_NBLK, _BLK, 1)
    w2 = weight.reshape(1, _C)
    b2 = bias.reshape(1, _C)

    out = pl.pallas_call(
        _body,
        grid=(2, _NBLK),
        in_specs=[
            pl.BlockSpec((1, 1, _BLK), lambda p, i: (i, 0, 0)),
            pl.BlockSpec((1, _BLK, 1), lambda p, i: (i, 0, 0)),
            pl.BlockSpec((_BLK, _C), lambda p, i: ((1 - p) * i, 0)),
            pl.BlockSpec((1, _C), lambda p, i: (0, 0)),
            pl.BlockSpec((1, _C), lambda p, i: (0, 0)),
        ],
        out_specs=pl.BlockSpec((_BLK, _C), lambda p, i: (p * i, 0)),
        out_shape=jax.ShapeDtypeStruct((_N, _C), jnp.float32),
        scratch_shapes=[
            pltpu.VMEM((_NI, _C), jnp.float32),
            pltpu.VMEM((_NI, _C), jnp.float32),
            pltpu.VMEM((_NI, 1), jnp.float32),
            pltpu.VMEM((_NI, 2 * _C), jnp.bfloat16),
            pltpu.VMEM((_N, _C), jnp.bfloat16),
        ],
    )(seg_row, seg_col, features, w2, b2)
    return out


# speedup vs baseline: 1.0072x; 1.0072x over previous
import jax
import jax.numpy as jnp
from jax import lax
from jax.experimental import pallas as pl
from jax.experimental.pallas import tpu as pltpu

_N = 32768
_C = 256
_G = 32
_CPG = _C // _G
_NI = 64
_EPS = 1e-5
_BLK = 4096
_NBLK = _N // _BLK
_HI = lax.Precision.HIGHEST


def _hilo(v):
    hi = v.astype(jnp.bfloat16)
    lo = (v - hi.astype(jnp.float32)).astype(jnp.bfloat16)
    return hi, lo


def _body(seg_row_ref, seg_col_ref, x_ref, w_ref, b_ref, o_ref,
          s1_acc, s2_acc, cnt_acc, tab, xsave):
    p = pl.program_id(0)
    i = pl.program_id(1)

    @pl.when(p == 0)
    def _stats():
        @pl.when(i == 0)
        def _init():
            s1_acc[...] = jnp.zeros_like(s1_acc)
            s2_acc[...] = jnp.zeros_like(s2_acc)
            cnt_acc[...] = jnp.zeros_like(cnt_acc)

        x = x_ref[...]
        xsave[pl.ds(i * _BLK, _BLK), :] = x
        xb = x.astype(jnp.bfloat16)
        seg = seg_row_ref[0]
        onehot = (
            lax.broadcasted_iota(jnp.int32, (_NI, _BLK), 0) == seg
        ).astype(jnp.bfloat16)
        s1_acc[...] += jnp.dot(onehot, xb, preferred_element_type=jnp.float32)
        s2_acc[...] += jnp.dot(onehot, xb * xb, preferred_element_type=jnp.float32)
        cnt_acc[...] += jnp.sum(
            onehot.astype(jnp.float32), axis=1, keepdims=True
        )

        @pl.when(i == _NBLK - 1)
        def _epilogue():
            s12 = jnp.concatenate([s1_acc[...], s2_acc[...]], axis=1)
            eg = (
                lax.broadcasted_iota(jnp.int32, (2 * _C, 2 * _G), 0) // _CPG
                == lax.broadcasted_iota(jnp.int32, (2 * _C, 2 * _G), 1)
            ).astype(jnp.float32)
            sg = jnp.dot(s12, eg, precision=_HI)
            cnt = jnp.maximum(cnt_acc[...] * float(_CPG), 1.0)
            mean = sg[:, :_G] / cnt
            var = sg[:, _G:] / cnt - mean * mean
            inv = lax.rsqrt(var + _EPS)
            rg = (
                lax.broadcasted_iota(jnp.int32, (_G, _C), 0)
                == lax.broadcasted_iota(jnp.int32, (_G, _C), 1) // _CPG
            ).astype(jnp.float32)
            inv_c = jnp.dot(inv, rg, precision=_HI)
            mean_c = jnp.dot(mean, rg, precision=_HI)
            scale = inv_c * w_ref[...]
            shift = b_ref[...] - mean_c * scale
            tab[...] = jnp.concatenate(
                [scale.astype(jnp.bfloat16), shift.astype(jnp.bfloat16)], axis=1
            )

    @pl.when(p == 1)
    def _norm():
        segc = seg_col_ref[0]
        onehot = (
            segc == lax.broadcasted_iota(jnp.int32, (_BLK, _NI), 1)
        ).astype(jnp.bfloat16)
        t = tab[...]
        sc_tok = jnp.dot(onehot, t[:, :_C], preferred_element_type=jnp.float32)
        sh_tok = jnp.dot(onehot, t[:, _C:], preferred_element_type=jnp.float32)
        o_ref[...] = xsave[pl.ds(i * _BLK, _BLK), :] * sc_tok + sh_tok


def kernel(features, ins_indices_batch, ins_ids, weight, bias):
    del ins_ids
    seg = ins_indices_batch.astype(jnp.int32)
    seg_row = seg.reshape(_NBLK, 1, _BLK)
    seg_col = seg.reshape(_NBLK, _BLK, 1)
    w2 = weight.reshape(1, _C)
    b2 = bias.reshape(1, _C)

    out = pl.pallas_call(
        _body,
        grid=(2, _NBLK),
        in_specs=[
            pl.BlockSpec((1, 1, _BLK), lambda p, i: (i, 0, 0)),
            pl.BlockSpec((1, _BLK, 1), lambda p, i: (i, 0, 0)),
            pl.BlockSpec((_BLK, _C), lambda p, i: ((1 - p) * i, 0)),
            pl.BlockSpec((1, _C), lambda p, i: (0, 0)),
            pl.BlockSpec((1, _C), lambda p, i: (0, 0)),
        ],
        out_specs=pl.BlockSpec((_BLK, _C), lambda p, i: (p * i, 0)),
        out_shape=jax.ShapeDtypeStruct((_N, _C), jnp.float32),
        scratch_shapes=[
            pltpu.VMEM((_NI, _C), jnp.float32),
            pltpu.VMEM((_NI, _C), jnp.float32),
            pltpu.VMEM((_NI, 1), jnp.float32),
            pltpu.VMEM((_NI, 2 * _C), jnp.bfloat16),
            pltpu.VMEM((_N, _C), jnp.float32),
        ],
    )(seg_row, seg_col, features, w2, b2)
    return out
